# Initial kernel scaffold; baseline (speedup 1.0000x reference)
import jax
import jax.numpy as jnp
from jax.experimental import pallas as pl


def kernel(x, embed):
    raise NotImplementedError("write your pallas kernel here")


# speedup vs baseline: 2.6826x; 2.6826x over previous
import functools

import jax
import jax.numpy as jnp
from jax.experimental import pallas as pl


def _vq_tile_kernel(x_ref, emb_ref, dist_ref, ind_ref, q_ref):
    xt = x_ref[0]
    emb = emb_ref[...]
    fe = jax.lax.dot_general(
        xt, emb, dimension_numbers=(((1,), (1,)), ((), ())),
        preferred_element_type=jnp.float32)
    f2 = jnp.sum(xt ** 2, axis=-1, keepdims=True)
    e2 = jnp.sum(emb ** 2, axis=-1)
    dist = -(f2 - 2.0 * fe + e2[None, :])
    dist_ref[0] = dist
    ind = jnp.argmax(dist, axis=-1)
    ind_ref[0, 0] = ind
    k_iota = jax.lax.broadcasted_iota(jnp.int32, dist.shape, 1)
    onehot = (k_iota == ind[:, None]).astype(jnp.float32)
    q_ref[0] = jax.lax.dot_general(
        onehot, emb, dimension_numbers=(((1,), (0,)), ((), ())),
        preferred_element_type=jnp.float32,
        precision=jax.lax.Precision.HIGHEST)


@functools.partial(jax.jit, static_argnames=())
def kernel(x, embed):
    H, K, D = embed.shape
    orig_shape = x.shape
    N = x.size // (H * D)
    TN = 1024
    G = N // TN
    xf = x.reshape(G, TN, D)
    emb2 = embed.reshape(K, D)

    dist, ind, q = pl.pallas_call(
        _vq_tile_kernel,
        grid=(G,),
        in_specs=[
            pl.BlockSpec((1, TN, D), lambda i: (i, 0, 0)),
            pl.BlockSpec((K, D), lambda i: (0, 0)),
        ],
        out_specs=[
            pl.BlockSpec((1, TN, K), lambda i: (i, 0, 0)),
            pl.BlockSpec((1, 1, TN), lambda i: (i, 0, 0)),
            pl.BlockSpec((1, TN, D), lambda i: (i, 0, 0)),
        ],
        out_shape=[
            jax.ShapeDtypeStruct((G, TN, K), jnp.float32),
            jax.ShapeDtypeStruct((G, 1, TN), jnp.int32),
            jax.ShapeDtypeStruct((G, TN, D), jnp.float32),
        ],
    )(xf, emb2)

    quantize = q.reshape(orig_shape)
    embed_ind = ind.reshape(orig_shape[:-1])
    dist = dist.reshape(H, N, K)
    return quantize, embed_ind, dist


# speedup vs baseline: 3.1748x; 1.1835x over previous
import functools

import jax
import jax.numpy as jnp
from jax import lax
from jax.experimental import pallas as pl
from jax.experimental.pallas import tpu as pltpu
from jax.experimental.pallas import tpu_sc as plsc

_TN = 1024


def _dist_argmax_kernel(x_ref, emb_ref, f2_ref, e2_ref, dist_ref, ind_ref):
    xt = x_ref[0]
    emb = emb_ref[...]
    fe = jax.lax.dot_general(
        xt, emb, dimension_numbers=(((1,), (1,)), ((), ())),
        preferred_element_type=jnp.float32)
    dist = -(f2_ref[0] - 2.0 * fe + e2_ref[...])
    dist_ref[0] = dist
    ind_ref[0, 0] = jnp.argmax(dist, axis=-1)


def _make_sc_gather(K, D, B):
    info = plsc.get_sparse_core_info()
    NC, NS, L = info.num_cores, info.num_subcores, info.num_lanes
    NW = NC * NS
    CH = 128
    BPW = B // NW
    NCHUNK = BPW // CH
    mesh = plsc.VectorSubcoreMesh(core_axis_name="c", subcore_axis_name="s")

    @functools.partial(
        pl.kernel, mesh=mesh,
        compiler_params=pltpu.CompilerParams(use_tc_tiling_on_sc=False),
        out_type=jax.ShapeDtypeStruct((B, D), jnp.float32),
        scratch_types=[
            pltpu.VMEM((NCHUNK, CH), jnp.int32),
            pltpu.VMEM((BPW, D), jnp.float32),
            pltpu.SemaphoreType.DMA,
        ],
    )
    def sc_gather(table_hbm, idx_hbm, out_hbm, idx_v, rows_v, sem):
        wid = lax.axis_index("s") * NC + lax.axis_index("c")
        pltpu.sync_copy(idx_hbm.at[pl.ds(wid * NCHUNK, NCHUNK)], idx_v)
        copies = []
        for j in range(NCHUNK):
            copies.append(pltpu.async_copy(
                table_hbm.at[idx_v.at[j]],
                rows_v.at[pl.ds(j * CH, CH)], sem))
        for c in copies:
            c.wait()
        pltpu.sync_copy(rows_v, out_hbm.at[pl.ds(wid * BPW, BPW)])

    return sc_gather


@jax.jit
def kernel(x, embed):
    H, K, D = embed.shape
    orig_shape = x.shape
    N = x.size // (H * D)
    G = N // _TN
    xf = x.reshape(G, _TN, D)
    emb2 = embed.reshape(K, D)
    flatten = x.reshape(H, -1, D)
    f2 = jnp.sum(flatten ** 2, axis=-1, keepdims=True).reshape(G, _TN, 1)
    e2 = jnp.sum(embed ** 2, axis=-1)

    dist, ind = pl.pallas_call(
        _dist_argmax_kernel,
        grid=(G,),
        in_specs=[
            pl.BlockSpec((1, _TN, D), lambda i: (i, 0, 0)),
            pl.BlockSpec((K, D), lambda i: (0, 0)),
            pl.BlockSpec((1, _TN, 1), lambda i: (i, 0, 0)),
            pl.BlockSpec((1, K), lambda i: (0, 0)),
        ],
        out_specs=[
            pl.BlockSpec((1, _TN, K), lambda i: (i, 0, 0)),
            pl.BlockSpec((1, 1, _TN), lambda i: (i, 0, 0)),
        ],
        out_shape=[
            jax.ShapeDtypeStruct((G, _TN, K), jnp.float32),
            jax.ShapeDtypeStruct((G, 1, _TN), jnp.int32),
        ],
    )(xf, emb2, f2, e2)

    ind_flat = ind.reshape(N // 128, 128)
    quantize = _make_sc_gather(K, D, N)(emb2, ind_flat)

    return (quantize.reshape(orig_shape),
            ind.reshape(orig_shape[:-1]),
            dist.reshape(H, N, K))


# speedup vs baseline: 3.2125x; 1.0119x over previous
import functools

import jax
import jax.numpy as jnp
from jax import lax
from jax.experimental import pallas as pl
from jax.experimental.pallas import tpu as pltpu
from jax.experimental.pallas import tpu_sc as plsc

_TN = 1024


def _dist_argmax_kernel(x_ref, emb_ref, e2_ref, dist_ref, ind_ref, ind2_ref):
    xt = x_ref[0]
    emb = emb_ref[...]
    fe = jax.lax.dot_general(
        xt, emb, dimension_numbers=(((1,), (1,)), ((), ())),
        preferred_element_type=jnp.float32)
    f2 = jnp.sum(xt ** 2, axis=-1, keepdims=True)
    dist = -(f2 - 2.0 * fe + e2_ref[...])
    dist_ref[0] = dist
    ind = jnp.argmax(dist, axis=-1)
    ind_ref[0, 0] = ind
    ind2_ref[...] = ind.reshape(ind2_ref.shape)


def _make_sc_gather(K, D, B):
    info = plsc.get_sparse_core_info()
    NC, NS = info.num_cores, info.num_subcores
    NW = NC * NS
    CH = 128
    BPW = B // NW
    NCHUNK = BPW // CH
    mesh = plsc.VectorSubcoreMesh(core_axis_name="c", subcore_axis_name="s")

    @functools.partial(
        pl.kernel, mesh=mesh,
        out_type=jax.ShapeDtypeStruct((B, 2 * D), jnp.float32),
        scratch_types=[
            pltpu.VMEM((NCHUNK, CH), jnp.int32),
            pltpu.VMEM((BPW, 2 * D), jnp.float32),
            pltpu.SemaphoreType.DMA,
        ],
    )
    def sc_gather(table_hbm, idx_hbm, out_hbm, idx_v, rows_v, sem):
        wid = lax.axis_index("s") * NC + lax.axis_index("c")
        pltpu.sync_copy(idx_hbm.at[pl.ds(wid * NCHUNK, NCHUNK)], idx_v)
        copies = []
        for j in range(NCHUNK):
            copies.append(pltpu.async_copy(
                table_hbm.at[idx_v.at[j]],
                rows_v.at[pl.ds(j * CH, CH)], sem))
        for c in copies:
            c.wait()
        pltpu.sync_copy(rows_v, out_hbm.at[pl.ds(wid * BPW, BPW)])

    return sc_gather


@jax.jit
def kernel(x, embed):
    H, K, D = embed.shape
    orig_shape = x.shape
    N = x.size // (H * D)
    G = N // _TN
    xf = x.reshape(G, _TN, D)
    emb2 = embed.reshape(K, D)
    e2 = jnp.sum(embed ** 2, axis=-1)

    dist, ind, ind2 = pl.pallas_call(
        _dist_argmax_kernel,
        grid=(G,),
        in_specs=[
            pl.BlockSpec((1, _TN, D), lambda i: (i, 0, 0)),
            pl.BlockSpec((K, D), lambda i: (0, 0)),
            pl.BlockSpec((1, K), lambda i: (0, 0)),
        ],
        out_specs=[
            pl.BlockSpec((1, _TN, K), lambda i: (i, 0, 0)),
            pl.BlockSpec((1, 1, _TN), lambda i: (i, 0, 0)),
            pl.BlockSpec((_TN // 128, 128), lambda i: (i, 0)),
        ],
        out_shape=[
            jax.ShapeDtypeStruct((G, _TN, K), jnp.float32),
            jax.ShapeDtypeStruct((G, 1, _TN), jnp.int32),
            jax.ShapeDtypeStruct((N // 128, 128), jnp.int32),
        ],
    )(xf, emb2, e2)

    table = jnp.concatenate(
        [emb2, jnp.zeros((K, 128 - D), jnp.float32)], axis=1)
    rows = _make_sc_gather(K, D, N)(table, ind2)
    quantize = rows[:, :D]

    return (quantize.reshape(orig_shape),
            ind.reshape(orig_shape[:-1]),
            dist.reshape(H, N, K))


# speedup vs baseline: 3.2874x; 1.0233x over previous
import jax
import jax.numpy as jnp
from jax.experimental import pallas as pl

_TN = 1024


def _vq_kernel(xt_ref, emb_ref, e2r_ref, e2c_ref, dist_ref, indt_ref, qt_ref):
    xt = xt_ref[0]
    emb = emb_ref[...]
    x2 = xt * xt
    fe = jax.lax.dot_general(
        xt, emb, dimension_numbers=(((0,), (1,)), ((), ())),
        preferred_element_type=jnp.float32)
    ones = jnp.ones((xt.shape[0], 1), jnp.float32)
    f2c = jax.lax.dot_general(
        x2, ones, dimension_numbers=(((0,), (0,)), ((), ())),
        preferred_element_type=jnp.float32)
    dist_ref[0] = -(f2c - 2.0 * fe + e2r_ref[...])
    feT = jax.lax.dot_general(
        emb, xt, dimension_numbers=(((1,), (0,)), ((), ())),
        preferred_element_type=jnp.float32)
    f2l = jnp.sum(x2, axis=0, keepdims=True)
    distT = -(f2l - 2.0 * feT + e2c_ref[...])
    indT = jnp.argmax(distT, axis=0)
    indt_ref[0, 0] = indT
    k_iota = jax.lax.broadcasted_iota(jnp.int32, distT.shape, 0)
    onehot = (k_iota == indT[None, :]).astype(jnp.float32)
    qt_ref[0] = jax.lax.dot_general(
        emb, onehot, dimension_numbers=(((0,), (0,)), ((), ())),
        preferred_element_type=jnp.float32,
        precision=jax.lax.Precision.HIGHEST)


@jax.jit
def kernel(x, embed):
    H, K, D = embed.shape
    orig_shape = x.shape
    N = x.size // (H * D)
    G = N // _TN
    xT = x.reshape(G, _TN, D).transpose(0, 2, 1)
    emb2 = embed.reshape(K, D)
    e2 = jnp.sum(embed ** 2, axis=-1)
    e2c = e2.reshape(K, 1)

    dist, indT, qT = pl.pallas_call(
        _vq_kernel,
        grid=(G,),
        in_specs=[
            pl.BlockSpec((1, D, _TN), lambda i: (i, 0, 0)),
            pl.BlockSpec((K, D), lambda i: (0, 0)),
            pl.BlockSpec((1, K), lambda i: (0, 0)),
            pl.BlockSpec((K, 1), lambda i: (0, 0)),
        ],
        out_specs=[
            pl.BlockSpec((1, _TN, K), lambda i: (i, 0, 0)),
            pl.BlockSpec((1, 1, _TN), lambda i: (i, 0, 0)),
            pl.BlockSpec((1, D, _TN), lambda i: (i, 0, 0)),
        ],
        out_shape=[
            jax.ShapeDtypeStruct((G, _TN, K), jnp.float32),
            jax.ShapeDtypeStruct((G, 1, _TN), jnp.int32),
            jax.ShapeDtypeStruct((G, D, _TN), jnp.float32),
        ],
    )(xT, emb2, e2, e2c)

    quantize = qT.transpose(0, 2, 1).reshape(orig_shape)
    return (quantize,
            indT.reshape(orig_shape[:-1]),
            dist.reshape(H, N, K))


# speedup vs baseline: 5.4721x; 1.6645x over previous
import jax
import jax.numpy as jnp
from jax.experimental import pallas as pl

_TN = 1024


def _vq_kernel(xt_ref, emb_ref, e2r_ref, e2c_ref, dist_ref, indt_ref, qt_ref):
    xt = xt_ref[0]
    emb = emb_ref[...]
    xt2 = xt + xt
    x2 = xt * xt
    fe2 = jax.lax.dot_general(
        xt2, emb, dimension_numbers=(((0,), (1,)), ((), ())),
        preferred_element_type=jnp.float32)
    ones = jnp.ones((xt.shape[0], 1), jnp.float32)
    f2c = jax.lax.dot_general(
        x2, ones, dimension_numbers=(((0,), (0,)), ((), ())),
        preferred_element_type=jnp.float32)
    dist_ref[0] = (fe2 - f2c) - e2r_ref[...]
    feT2 = jax.lax.dot_general(
        emb, xt2, dimension_numbers=(((1,), (0,)), ((), ())),
        preferred_element_type=jnp.float32)
    f2l = jnp.sum(x2, axis=0, keepdims=True)
    distT = (feT2 - f2l) - e2c_ref[...]
    indT = jnp.argmax(distT, axis=0)
    indt_ref[0, 0] = indT
    k_iota = jax.lax.broadcasted_iota(jnp.int32, distT.shape, 0)
    onehot = (k_iota == indT[None, :]).astype(jnp.float32)
    emb_hi = emb.astype(jnp.bfloat16).astype(jnp.float32)
    emb_lo = emb - emb_hi
    dn = (((0,), (0,)), ((), ()))
    qt_ref[0] = (
        jax.lax.dot_general(emb_hi, onehot, dimension_numbers=dn,
                            preferred_element_type=jnp.float32)
        + jax.lax.dot_general(emb_lo, onehot, dimension_numbers=dn,
                              preferred_element_type=jnp.float32))


@jax.jit
def kernel(x, embed):
    H, K, D = embed.shape
    orig_shape = x.shape
    N = x.size // (H * D)
    G = N // _TN
    xT = x.reshape(G, _TN, D).transpose(0, 2, 1)
    emb2 = embed.reshape(K, D)
    e2 = jnp.sum(embed ** 2, axis=-1)
    e2c = e2.reshape(K, 1)

    dist, indT, qT = pl.pallas_call(
        _vq_kernel,
        grid=(G,),
        in_specs=[
            pl.BlockSpec((1, D, _TN), lambda i: (i, 0, 0)),
            pl.BlockSpec((K, D), lambda i: (0, 0)),
            pl.BlockSpec((1, K), lambda i: (0, 0)),
            pl.BlockSpec((K, 1), lambda i: (0, 0)),
        ],
        out_specs=[
            pl.BlockSpec((1, _TN, K), lambda i: (i, 0, 0)),
            pl.BlockSpec((1, 1, _TN), lambda i: (i, 0, 0)),
            pl.BlockSpec((1, D, _TN), lambda i: (i, 0, 0)),
        ],
        out_shape=[
            jax.ShapeDtypeStruct((G, _TN, K), jnp.float32),
            jax.ShapeDtypeStruct((G, 1, _TN), jnp.int32),
            jax.ShapeDtypeStruct((G, D, _TN), jnp.float32),
        ],
    )(xT, emb2, e2, e2c)

    quantize = qT.transpose(0, 2, 1).reshape(orig_shape)
    return (quantize,
            indT.reshape(orig_shape[:-1]),
            dist.reshape(H, N, K))


# speedup vs baseline: 5.6994x; 1.0415x over previous
import jax
import jax.numpy as jnp
from jax.experimental import pallas as pl
from jax.experimental.pallas import tpu as pltpu

_TN = 1024


def _vq_kernel(xt_ref, embt_ref, hilo_ref, e2r_ref, e2c_ref, f2r_ref, f2c_ref,
               dist_ref, indt_ref, qt_ref):
    xt = xt_ref[0]
    embt = embt_ref[...]
    xt2 = xt + xt
    cdim = (((0,), (0,)), ((), ()))
    fe2 = jax.lax.dot_general(
        xt2, embt, dimension_numbers=cdim,
        preferred_element_type=jnp.float32)
    dist_ref[0] = (fe2 - f2c_ref[0]) - e2r_ref[...]
    feT2 = jax.lax.dot_general(
        embt, xt2, dimension_numbers=cdim,
        preferred_element_type=jnp.float32)
    distT = (feT2 - f2r_ref[0]) - e2c_ref[...]
    indT = jnp.argmax(distT, axis=0)
    indt_ref[0, 0] = indT
    k_iota = jax.lax.broadcasted_iota(jnp.int32, distT.shape, 0)
    onehot = (k_iota == indT[None, :]).astype(jnp.float32)
    qt2 = jax.lax.dot_general(
        hilo_ref[...], onehot, dimension_numbers=cdim,
        preferred_element_type=jnp.float32)
    D = xt.shape[0]
    qt_ref[0] = qt2[:D] + qt2[D:]


@jax.jit
def kernel(x, embed):
    H, K, D = embed.shape
    orig_shape = x.shape
    N = x.size // (H * D)
    G = N // _TN
    xT = x.reshape(G, _TN, D).transpose(0, 2, 1)
    emb2 = embed.reshape(K, D)
    embT = emb2.T
    emb_hi = emb2.astype(jnp.bfloat16).astype(jnp.float32)
    hilo = jnp.concatenate([emb_hi, emb2 - emb_hi], axis=1)
    e2 = jnp.sum(embed ** 2, axis=-1)
    e2c = e2.reshape(K, 1)
    flatten = x.reshape(H, -1, D)
    f2 = jnp.sum(flatten ** 2, axis=-1)
    f2r = f2.reshape(G, 1, _TN)
    f2c = f2.reshape(G, _TN, 1)

    dist, indT, qT = pl.pallas_call(
        _vq_kernel,
        grid=(G,),
        in_specs=[
            pl.BlockSpec((1, D, _TN), lambda i: (i, 0, 0)),
            pl.BlockSpec((D, K), lambda i: (0, 0)),
            pl.BlockSpec((K, 2 * D), lambda i: (0, 0)),
            pl.BlockSpec((1, K), lambda i: (0, 0)),
            pl.BlockSpec((K, 1), lambda i: (0, 0)),
            pl.BlockSpec((1, 1, _TN), lambda i: (i, 0, 0)),
            pl.BlockSpec((1, _TN, 1), lambda i: (i, 0, 0)),
        ],
        out_specs=[
            pl.BlockSpec((1, _TN, K), lambda i: (i, 0, 0)),
            pl.BlockSpec((1, 1, _TN), lambda i: (i, 0, 0)),
            pl.BlockSpec((1, D, _TN), lambda i: (i, 0, 0)),
        ],
        out_shape=[
            jax.ShapeDtypeStruct((G, _TN, K), jnp.float32),
            jax.ShapeDtypeStruct((G, 1, _TN), jnp.int32),
            jax.ShapeDtypeStruct((G, D, _TN), jnp.float32),
        ],
        compiler_params=pltpu.CompilerParams(
            dimension_semantics=("arbitrary",)),
    )(xT, embT, hilo, e2, e2c, f2r, f2c)

    quantize = qT.transpose(0, 2, 1).reshape(orig_shape)
    return (quantize,
            indT.reshape(orig_shape[:-1]),
            dist.reshape(H, N, K))
